# baseline (speedup 1.0000x reference)
import functools

import jax
import jax.numpy as jnp
from jax import lax
from jax.experimental import pallas as pl
from jax.experimental.pallas import tpu as pltpu
from jax.experimental.pallas import tpu_sc as plsc

VOCAB = 1000000
EMB = 64
ROWS = 4096
SEQ = 200
TOTAL = ROWS * SEQ
NC = 2
NS = 16
NW = NC * NS
PER_W = TOTAL // NW
CHUNK = 128
NCHUNK = PER_W // CHUNK
NB = 4
NGROUP = NCHUNK // NB
SCALE = 1.0 / 64.0


@functools.partial(
    pl.kernel,
    mesh=plsc.VectorSubcoreMesh(core_axis_name="c", subcore_axis_name="s"),
    out_type=jax.ShapeDtypeStruct((TOTAL, EMB), jnp.float32),
    compiler_params=pltpu.CompilerParams(use_tc_tiling_on_sc=False),
    scratch_types=[
        pltpu.VMEM((PER_W,), jnp.int32),
        pltpu.VMEM((CHUNK, EMB), jnp.float32),
        pltpu.VMEM((CHUNK, EMB), jnp.float32),
        pltpu.VMEM((CHUNK, EMB), jnp.float32),
        pltpu.VMEM((CHUNK, EMB), jnp.float32),
        pltpu.SemaphoreType.DMA,
        pltpu.SemaphoreType.DMA,
        pltpu.SemaphoreType.DMA,
        pltpu.SemaphoreType.DMA,
        pltpu.SemaphoreType.DMA,
        pltpu.SemaphoreType.DMA,
        pltpu.SemaphoreType.DMA,
        pltpu.SemaphoreType.DMA,
    ],
)
def _emb_lookup(idx_hbm, table_hbm, out_hbm, idx_v,
                b0, b1, b2, b3, g0, g1, g2, g3, o0, o1, o2, o3):
    bufs = (b0, b1, b2, b3)
    gsems = (g0, g1, g2, g3)
    osems = (o0, o1, o2, o3)
    wid = lax.axis_index("s") * NC + lax.axis_index("c")
    base = wid * PER_W
    pltpu.sync_copy(idx_hbm.at[pl.ds(base, PER_W)], idx_v)

    def start_gather(j, k):
        pltpu.async_copy(
            table_hbm.at[idx_v.at[pl.ds(j * CHUNK, CHUNK)]], bufs[k], gsems[k])

    def wait_gather(k):
        pltpu.make_async_copy(
            table_hbm.at[idx_v.at[pl.ds(0, CHUNK)]], bufs[k], gsems[k]).wait()

    def wait_store(k):
        pltpu.make_async_copy(
            bufs[k], out_hbm.at[pl.ds(base, CHUNK)], osems[k]).wait()

    for k in range(NB):
        start_gather(k, k)

    def group_body(g, carry):
        for k in range(NB):
            j = g * NB + k
            wait_gather(k)

            def row_body(i, c, buf=bufs[k]):
                for t in range(EMB // 16):
                    sl = pl.ds(t * 16, 16)
                    buf[i, sl] = buf[i, sl] * SCALE
                return c

            lax.fori_loop(0, CHUNK, row_body, 0, unroll=2)
            pltpu.async_copy(
                bufs[k], out_hbm.at[pl.ds(base + j * CHUNK, CHUNK)], osems[k])

        @pl.when(g + 1 < NGROUP)
        def _prefetch():
            for k in range(NB):
                wait_store(k)
                start_gather((g + 1) * NB + k, k)

        return carry

    lax.fori_loop(0, NGROUP, group_body, 0)
    for k in range(NB):
        wait_store(k)


def kernel(inp, table):
    flat = inp.reshape(TOTAL)
    out = _emb_lookup(flat, table)
    return out.reshape(ROWS, SEQ, EMB)
